# baseline (speedup 1.0000x reference)
import functools

import jax
import jax.numpy as jnp
from jax.experimental import pallas as pl
from jax.experimental.pallas import tpu as pltpu

_TM = 200


def _sage_kernel(x_ref, wrel_ref, adj_ref, wroot_ref, b_ref, out_ref, y_ref):
    i = pl.program_id(0)

    @pl.when(i == 0)
    def _compute_y():
        y_ref[...] = jnp.dot(
            x_ref[...].astype(jnp.bfloat16),
            wrel_ref[...],
            preferred_element_type=jnp.float32,
        ).astype(jnp.bfloat16)

    adj = adj_ref[...]
    deg = jnp.maximum(jnp.sum(adj, axis=1, keepdims=True), 1.0)
    agg = jnp.dot(
        adj.astype(jnp.bfloat16), y_ref[...], preferred_element_type=jnp.float32
    )
    root = jnp.dot(
        x_ref[pl.ds(i * _TM, _TM), :],
        wroot_ref[...],
        preferred_element_type=jnp.float32,
    )
    out_ref[...] = agg / deg + root + b_ref[...]


@jax.jit
def kernel(x, adj, W_rel, b_rel, W_root):
    N, C = x.shape
    H = W_rel.shape[0]
    return pl.pallas_call(
        _sage_kernel,
        grid=(N // _TM,),
        in_specs=[
            pl.BlockSpec((N, C), lambda i: (0, 0)),
            pl.BlockSpec((C, H), lambda i: (0, 0)),
            pl.BlockSpec((_TM, N), lambda i: (i, 0)),
            pl.BlockSpec((C, H), lambda i: (0, 0)),
            pl.BlockSpec((1, H), lambda i: (0, 0)),
        ],
        out_specs=pl.BlockSpec((_TM, H), lambda i: (i, 0)),
        out_shape=jax.ShapeDtypeStruct((N, H), jnp.float32),
        scratch_shapes=[pltpu.VMEM((N, H), jnp.bfloat16)],
    )(x, W_rel.T.astype(jnp.bfloat16), adj, W_root.T, b_rel.reshape(1, H))


# speedup vs baseline: 1.0506x; 1.0506x over previous
import jax
import jax.numpy as jnp
from jax.experimental import pallas as pl
from jax.experimental.pallas import tpu as pltpu

_TM = 400


def _sage_kernel(x_ref, wrel_ref, adj_ref, wroot_ref, b_ref, out_ref, y_ref):
    i = pl.program_id(0)

    @pl.when(i == 0)
    def _compute_y():
        y_ref[...] = jnp.dot(
            x_ref[...].astype(jnp.bfloat16),
            wrel_ref[...],
            preferred_element_type=jnp.float32,
        ).astype(jnp.bfloat16)

    adj = adj_ref[...]
    deg = jnp.maximum(jnp.sum(adj, axis=1, keepdims=True), 1.0)
    agg = jnp.dot(
        adj.astype(jnp.bfloat16), y_ref[...], preferred_element_type=jnp.float32
    )
    root = jnp.dot(
        x_ref[pl.ds(i * _TM, _TM), :],
        wroot_ref[...],
        preferred_element_type=jnp.float32,
    )
    out_ref[...] = agg / deg + root + b_ref[...]


@jax.jit
def kernel(x, adj, W_rel, b_rel, W_root):
    N, C = x.shape
    H = W_rel.shape[0]
    return pl.pallas_call(
        _sage_kernel,
        grid=(N // _TM,),
        in_specs=[
            pl.BlockSpec((N, C), lambda i: (0, 0)),
            pl.BlockSpec((C, H), lambda i: (0, 0)),
            pl.BlockSpec((_TM, N), lambda i: (i, 0)),
            pl.BlockSpec((C, H), lambda i: (0, 0)),
            pl.BlockSpec((1, H), lambda i: (0, 0)),
        ],
        out_specs=pl.BlockSpec((_TM, H), lambda i: (i, 0)),
        out_shape=jax.ShapeDtypeStruct((N, H), jnp.float32),
        scratch_shapes=[pltpu.VMEM((N, H), jnp.bfloat16)],
    )(x, W_rel.T.astype(jnp.bfloat16), adj, W_root.T, b_rel.reshape(1, H))
